# baseline (speedup 1.0000x reference)
import functools

import jax
import jax.numpy as jnp
from jax import lax
from jax.experimental import pallas as pl
from jax.experimental.pallas import tpu as pltpu
from jax.experimental.pallas import tpu_sc as plsc

N_ATOMS = 10000
N_BONDS = 320000
MAX_NB = 32
ATOM_FDIM = 128
BOND_FDIM = 144
H = 128
N_MOLS = 100
MOL_SIZE = 100

NC = 2
NS = 16
NW = NC * NS
L = 16

A_PAD = 10240


def _worker_id():
    return lax.axis_index("s") * NC + lax.axis_index("c")


def _make_gather_summax(n_src):
    CH = 4
    RW = A_PAD // NW
    n_chunks = RW // CH
    NB = 4
    n_outer = n_chunks // NB
    mesh = plsc.VectorSubcoreMesh(core_axis_name="c", subcore_axis_name="s")

    @functools.partial(
        pl.kernel,
        name=f"gsm_{n_src}",
        out_type=jax.ShapeDtypeStruct((A_PAD, H), jnp.float32),
        mesh=mesh,
        scratch_types=(
            [pltpu.VMEM((RW * MAX_NB,), jnp.int32)]
            + [pltpu.VMEM((RW, H), jnp.float32)]
            + [pltpu.VMEM((CH * MAX_NB, H), jnp.float32) for _ in range(NB)]
            + [pltpu.SemaphoreType.DMA for _ in range(NB)]
        ),
    )
    def k(table_hbm, idx_hbm, out_hbm, idx_v, out_all, *bufs):
        rows = bufs[:NB]
        gsem = bufs[NB:2 * NB]
        wid = _worker_id()
        base = wid * RW

        pltpu.sync_copy(idx_hbm.at[pl.ds(base * MAX_NB, RW * MAX_NB)], idx_v)

        def start_gather(c, b):
            pltpu.async_copy(
                table_hbm.at[idx_v.at[pl.ds(c * CH * MAX_NB, CH * MAX_NB)]],
                rows[b], gsem[b])

        for b in range(NB):
            start_gather(b, b)

        def outer(i, carry):
            c0 = i * NB
            for b in range(NB):
                c = c0 + b
                pltpu.make_async_copy(
                    table_hbm.at[pl.ds(0, CH * MAX_NB)], rows[b], gsem[b]
                ).wait()

                def per_atom(a, c2):
                    r0 = a * MAX_NB
                    nl = H // L
                    v0 = [rows[b][r0, pl.ds(l * L, L)] for l in range(nl)]
                    s_acc = list(v0)
                    m_acc = list(v0)
                    for r in range(1, MAX_NB):
                        vn = [rows[b][r0 + r, pl.ds(l * L, L)]
                              for l in range(nl)]
                        s_acc = [s_acc[l] + vn[l] for l in range(nl)]
                        m_acc = [jnp.maximum(m_acc[l], vn[l])
                                 for l in range(nl)]
                    for l in range(nl):
                        out_all[c * CH + a, pl.ds(l * L, L)] = (
                            s_acc[l] * m_acc[l])
                    return c2

                lax.fori_loop(0, CH, per_atom, 0)

                @pl.when(i + 1 < n_outer)
                def _():
                    start_gather(c - c0 + (i + 1) * NB, b)
            return carry

        lax.fori_loop(0, n_outer, outer, 0)
        pltpu.sync_copy(out_all, out_hbm.at[pl.ds(base, RW)])

    return k


def _make_dual_gather_sub(n_atab):
    CB = 40
    RW = N_BONDS // NW
    n_chunks = RW // CB
    NB = 5
    n_outer = n_chunks // NB
    mesh = plsc.VectorSubcoreMesh(core_axis_name="c", subcore_axis_name="s")

    @functools.partial(
        pl.kernel,
        name="dual_gather_sub",
        out_type=jax.ShapeDtypeStruct((N_BONDS, H), jnp.float32),
        mesh=mesh,
        scratch_types=(
            [pltpu.VMEM((RW,), jnp.int32) for _ in range(2)]
            + [pltpu.VMEM((CB, H), jnp.float32) for _ in range(3 * NB)]
            + [pltpu.SemaphoreType.DMA for _ in range(3 * NB)]
        ),
    )
    def k(amsg_hbm, msg_hbm, b2a_hbm, b2revb_hbm, out_hbm, ia_v, ir_v, *bufs):
        ra = bufs[0:NB]
        rr = bufs[NB:2 * NB]
        dv = bufs[2 * NB:3 * NB]
        sa = bufs[3 * NB:4 * NB]
        sr = bufs[4 * NB:5 * NB]
        sw = bufs[5 * NB:6 * NB]
        wid = _worker_id()
        base = wid * RW

        pltpu.sync_copy(b2a_hbm.at[pl.ds(base, RW)], ia_v)
        pltpu.sync_copy(b2revb_hbm.at[pl.ds(base, RW)], ir_v)
        for b in range(NB):
            pltpu.async_copy(amsg_hbm.at[ia_v.at[pl.ds(b * CB, CB)]], ra[b], sa[b])
            pltpu.async_copy(msg_hbm.at[ir_v.at[pl.ds(b * CB, CB)]], rr[b], sr[b])

        def outer(i, carry):
            c0 = i * NB
            for b in range(NB):
                c = c0 + b
                pltpu.make_async_copy(
                    amsg_hbm.at[pl.ds(0, CB)], ra[b], sa[b]).wait()
                pltpu.make_async_copy(
                    msg_hbm.at[pl.ds(0, CB)], rr[b], sr[b]).wait()

                @pl.when(i > 0)
                def _():
                    pltpu.make_async_copy(
                        dv[b], out_hbm.at[pl.ds(0, CB)], sw[b]).wait()

                def per_row(j, c2):
                    for l in range(H // L):
                        off = l * L
                        dv[b][j, pl.ds(off, L)] = (
                            ra[b][j, pl.ds(off, L)] - rr[b][j, pl.ds(off, L)])
                    return c2

                lax.fori_loop(0, CB, per_row, 0)
                pltpu.async_copy(dv[b], out_hbm.at[pl.ds(base + c * CB, CB)],
                                 sw[b])

                @pl.when(i + 1 < n_outer)
                def _():
                    cn = c - c0 + (i + 1) * NB
                    pltpu.async_copy(amsg_hbm.at[ia_v.at[pl.ds(cn * CB, CB)]],
                                     ra[b], sa[b])
                    pltpu.async_copy(msg_hbm.at[ir_v.at[pl.ds(cn * CB, CB)]],
                                     rr[b], sr[b])
            return carry

        lax.fori_loop(0, n_outer, outer, 0)
        for b in range(NB):
            pltpu.make_async_copy(dv[b], out_hbm.at[pl.ds(0, CB)],
                                  sw[b]).wait()

    return k


def _mm_input(x, W, bm):
    M, K = x.shape

    def body(x_ref, w_ref, inp_ref, msg_ref):
        acc = jnp.dot(x_ref[...], w_ref[...], preferred_element_type=jnp.float32)
        inp_ref[...] = acc
        msg_ref[...] = jnp.maximum(acc, 0.0)

    return pl.pallas_call(
        body,
        grid=(M // bm,),
        in_specs=[pl.BlockSpec((bm, K), lambda i: (i, 0)),
                  pl.BlockSpec((K, H), lambda i: (0, 0))],
        out_specs=[pl.BlockSpec((bm, H), lambda i: (i, 0)),
                   pl.BlockSpec((bm, H), lambda i: (i, 0))],
        out_shape=[jax.ShapeDtypeStruct((M, H), jnp.float32)] * 2,
    )(x, W)


def _iter_step(d, inp, Wh, bm):
    M = d.shape[0]

    def body(d_ref, i_ref, w_ref, o_ref):
        acc = jnp.dot(d_ref[...], w_ref[...], preferred_element_type=jnp.float32)
        o_ref[...] = jnp.maximum(i_ref[...] + acc, 0.0)

    return pl.pallas_call(
        body,
        grid=(M // bm,),
        in_specs=[pl.BlockSpec((bm, H), lambda i: (i, 0)),
                  pl.BlockSpec((bm, H), lambda i: (i, 0)),
                  pl.BlockSpec((H, H), lambda i: (0, 0))],
        out_specs=pl.BlockSpec((bm, H), lambda i: (i, 0)),
        out_shape=jax.ShapeDtypeStruct((M, H), jnp.float32),
    )(d, inp, Wh)


def _out_layer(fa, am, W_o, b_o, bm):
    M = fa.shape[0]
    W1 = W_o[:ATOM_FDIM]
    W2 = W_o[ATOM_FDIM:]
    b2d = b_o.reshape(1, H)

    def body(fa_ref, am_ref, w1_ref, w2_ref, b_ref, o_ref):
        acc = jnp.dot(fa_ref[...], w1_ref[...], preferred_element_type=jnp.float32)
        acc = acc + jnp.dot(am_ref[...], w2_ref[...], preferred_element_type=jnp.float32)
        o_ref[...] = jnp.maximum(acc + b_ref[...], 0.0)

    return pl.pallas_call(
        body,
        grid=(M // bm,),
        in_specs=[pl.BlockSpec((bm, ATOM_FDIM), lambda i: (i, 0)),
                  pl.BlockSpec((bm, H), lambda i: (i, 0)),
                  pl.BlockSpec((ATOM_FDIM, H), lambda i: (0, 0)),
                  pl.BlockSpec((H, H), lambda i: (0, 0)),
                  pl.BlockSpec((1, H), lambda i: (0, 0))],
        out_specs=pl.BlockSpec((bm, H), lambda i: (i, 0)),
        out_shape=jax.ShapeDtypeStruct((M, H), jnp.float32),
    )(fa, am, W1, W2, b2d)


def _pad_idx(ix):
    return jnp.pad(ix, ((0, A_PAD - N_ATOMS), (0, 0))).reshape(-1)


def kernel(f_atoms, f_bonds, a2b, b2a, b2revb, a2a,
           W_i_a, W_h_a_0, W_h_a_1, W_o_a, b_o_a,
           W_i_b, W_h_b_0, W_h_b_1, W_o_b, b_o_b):
    a2a_f = _pad_idx(a2a.astype(jnp.int32))
    a2b_f = _pad_idx(a2b.astype(jnp.int32))
    b2a32 = b2a.astype(jnp.int32)
    b2revb32 = b2revb.astype(jnp.int32)

    gsm_atom = _make_gather_summax(N_ATOMS)
    gsm_bond = _make_gather_summax(N_BONDS)
    dual = _make_dual_gather_sub(N_ATOMS)

    inp_a, msg = _mm_input(f_atoms, W_i_a, 400)
    for Wh in (W_h_a_0, W_h_a_1):
        am = gsm_atom(msg, a2a_f)[:N_ATOMS]
        msg = _iter_step(am, inp_a, Wh, 400)
    am = gsm_atom(msg, a2a_f)[:N_ATOMS]
    atom_h = _out_layer(f_atoms, am, W_o_a, b_o_a, 400)
    atom_vecs = atom_h.reshape(N_MOLS, MOL_SIZE, H)

    inp_b, msg = _mm_input(f_bonds, W_i_b, 2560)
    for Wh in (W_h_b_0, W_h_b_1):
        am = gsm_bond(msg, a2b_f)[:N_ATOMS]
        d = dual(am, msg, b2a32, b2revb32)
        msg = _iter_step(d, inp_b, Wh, 2560)
    am = gsm_bond(msg, a2b_f)[:N_ATOMS]
    bond_h = _out_layer(f_atoms, am, W_o_b, b_o_b, 400)
    bond_vecs = bond_h.reshape(N_MOLS, MOL_SIZE, H)

    mask = jnp.ones((N_MOLS, MOL_SIZE), dtype=jnp.float32)
    return (atom_vecs, mask, bond_vecs, mask)


# speedup vs baseline: 1.0011x; 1.0011x over previous
import functools

import jax
import jax.numpy as jnp
from jax import lax
from jax.experimental import pallas as pl
from jax.experimental.pallas import tpu as pltpu
from jax.experimental.pallas import tpu_sc as plsc

N_ATOMS = 10000
N_BONDS = 320000
MAX_NB = 32
ATOM_FDIM = 128
BOND_FDIM = 144
H = 128
N_MOLS = 100
MOL_SIZE = 100

NC = 2
NS = 16
NW = NC * NS
L = 16

A_PAD = 10240


def _worker_id():
    return lax.axis_index("s") * NC + lax.axis_index("c")


def _make_gather_summax(n_src):
    CH = 2
    RW = A_PAD // NW
    n_chunks = RW // CH
    NB = 5
    n_outer = n_chunks // NB
    mesh = plsc.VectorSubcoreMesh(core_axis_name="c", subcore_axis_name="s")

    @functools.partial(
        pl.kernel,
        name=f"gsm_{n_src}",
        out_type=jax.ShapeDtypeStruct((A_PAD, H), jnp.float32),
        mesh=mesh,
        scratch_types=(
            [pltpu.VMEM((RW * MAX_NB,), jnp.int32)]
            + [pltpu.VMEM((RW, H), jnp.float32)]
            + [pltpu.VMEM((CH * MAX_NB, H), jnp.float32) for _ in range(NB)]
            + [pltpu.SemaphoreType.DMA for _ in range(NB)]
        ),
    )
    def k(table_hbm, idx_hbm, out_hbm, idx_v, out_all, *bufs):
        rows = bufs[:NB]
        gsem = bufs[NB:2 * NB]
        wid = _worker_id()
        base = wid * RW

        pltpu.sync_copy(idx_hbm.at[pl.ds(base * MAX_NB, RW * MAX_NB)], idx_v)

        def start_gather(c, b):
            pltpu.async_copy(
                table_hbm.at[idx_v.at[pl.ds(c * CH * MAX_NB, CH * MAX_NB)]],
                rows[b], gsem[b])

        for b in range(NB):
            start_gather(b, b)

        def outer(i, carry):
            c0 = i * NB
            for b in range(NB):
                c = c0 + b
                pltpu.make_async_copy(
                    table_hbm.at[pl.ds(0, CH * MAX_NB)], rows[b], gsem[b]
                ).wait()

                def per_atom(a, c2):
                    r0 = a * MAX_NB
                    nl = H // L
                    v0 = [rows[b][r0, pl.ds(l * L, L)] for l in range(nl)]
                    s_acc = list(v0)
                    m_acc = list(v0)
                    for r in range(1, MAX_NB):
                        vn = [rows[b][r0 + r, pl.ds(l * L, L)]
                              for l in range(nl)]
                        s_acc = [s_acc[l] + vn[l] for l in range(nl)]
                        m_acc = [jnp.maximum(m_acc[l], vn[l])
                                 for l in range(nl)]
                    for l in range(nl):
                        out_all[c * CH + a, pl.ds(l * L, L)] = (
                            s_acc[l] * m_acc[l])
                    return c2

                lax.fori_loop(0, CH, per_atom, 0)

                @pl.when(i + 1 < n_outer)
                def _():
                    start_gather(c - c0 + (i + 1) * NB, b)
            return carry

        lax.fori_loop(0, n_outer, outer, 0)
        pltpu.sync_copy(out_all, out_hbm.at[pl.ds(base, RW)])

    return k


def _make_dual_gather_sub(n_atab):
    CB = 40
    RW = N_BONDS // NW
    n_chunks = RW // CB
    NB = 5
    n_outer = n_chunks // NB
    mesh = plsc.VectorSubcoreMesh(core_axis_name="c", subcore_axis_name="s")

    @functools.partial(
        pl.kernel,
        name="dual_gather_sub",
        out_type=jax.ShapeDtypeStruct((N_BONDS, H), jnp.float32),
        mesh=mesh,
        scratch_types=(
            [pltpu.VMEM((RW,), jnp.int32) for _ in range(2)]
            + [pltpu.VMEM((CB, H), jnp.float32) for _ in range(3 * NB)]
            + [pltpu.SemaphoreType.DMA for _ in range(3 * NB)]
        ),
    )
    def k(amsg_hbm, msg_hbm, b2a_hbm, b2revb_hbm, out_hbm, ia_v, ir_v, *bufs):
        ra = bufs[0:NB]
        rr = bufs[NB:2 * NB]
        dv = bufs[2 * NB:3 * NB]
        sa = bufs[3 * NB:4 * NB]
        sr = bufs[4 * NB:5 * NB]
        sw = bufs[5 * NB:6 * NB]
        wid = _worker_id()
        base = wid * RW

        pltpu.sync_copy(b2a_hbm.at[pl.ds(base, RW)], ia_v)
        pltpu.sync_copy(b2revb_hbm.at[pl.ds(base, RW)], ir_v)
        for b in range(NB):
            pltpu.async_copy(amsg_hbm.at[ia_v.at[pl.ds(b * CB, CB)]], ra[b], sa[b])
            pltpu.async_copy(msg_hbm.at[ir_v.at[pl.ds(b * CB, CB)]], rr[b], sr[b])

        def outer(i, carry):
            c0 = i * NB
            for b in range(NB):
                c = c0 + b
                pltpu.make_async_copy(
                    amsg_hbm.at[pl.ds(0, CB)], ra[b], sa[b]).wait()
                pltpu.make_async_copy(
                    msg_hbm.at[pl.ds(0, CB)], rr[b], sr[b]).wait()

                @pl.when(i > 0)
                def _():
                    pltpu.make_async_copy(
                        dv[b], out_hbm.at[pl.ds(0, CB)], sw[b]).wait()

                def per_row(j, c2):
                    for l in range(H // L):
                        off = l * L
                        dv[b][j, pl.ds(off, L)] = (
                            ra[b][j, pl.ds(off, L)] - rr[b][j, pl.ds(off, L)])
                    return c2

                lax.fori_loop(0, CB, per_row, 0)
                pltpu.async_copy(dv[b], out_hbm.at[pl.ds(base + c * CB, CB)],
                                 sw[b])

                @pl.when(i + 1 < n_outer)
                def _():
                    cn = c - c0 + (i + 1) * NB
                    pltpu.async_copy(amsg_hbm.at[ia_v.at[pl.ds(cn * CB, CB)]],
                                     ra[b], sa[b])
                    pltpu.async_copy(msg_hbm.at[ir_v.at[pl.ds(cn * CB, CB)]],
                                     rr[b], sr[b])
            return carry

        lax.fori_loop(0, n_outer, outer, 0)
        for b in range(NB):
            pltpu.make_async_copy(dv[b], out_hbm.at[pl.ds(0, CB)],
                                  sw[b]).wait()

    return k


def _mm_input(x, W, bm):
    M, K = x.shape

    def body(x_ref, w_ref, inp_ref, msg_ref):
        acc = jnp.dot(x_ref[...], w_ref[...], preferred_element_type=jnp.float32)
        inp_ref[...] = acc
        msg_ref[...] = jnp.maximum(acc, 0.0)

    return pl.pallas_call(
        body,
        grid=(M // bm,),
        in_specs=[pl.BlockSpec((bm, K), lambda i: (i, 0)),
                  pl.BlockSpec((K, H), lambda i: (0, 0))],
        out_specs=[pl.BlockSpec((bm, H), lambda i: (i, 0)),
                   pl.BlockSpec((bm, H), lambda i: (i, 0))],
        out_shape=[jax.ShapeDtypeStruct((M, H), jnp.float32)] * 2,
    )(x, W)


def _iter_step(d, inp, Wh, bm):
    M = d.shape[0]

    def body(d_ref, i_ref, w_ref, o_ref):
        acc = jnp.dot(d_ref[...], w_ref[...], preferred_element_type=jnp.float32)
        o_ref[...] = jnp.maximum(i_ref[...] + acc, 0.0)

    return pl.pallas_call(
        body,
        grid=(M // bm,),
        in_specs=[pl.BlockSpec((bm, H), lambda i: (i, 0)),
                  pl.BlockSpec((bm, H), lambda i: (i, 0)),
                  pl.BlockSpec((H, H), lambda i: (0, 0))],
        out_specs=pl.BlockSpec((bm, H), lambda i: (i, 0)),
        out_shape=jax.ShapeDtypeStruct((M, H), jnp.float32),
    )(d, inp, Wh)


def _out_layer(fa, am, W_o, b_o, bm):
    M = fa.shape[0]
    W1 = W_o[:ATOM_FDIM]
    W2 = W_o[ATOM_FDIM:]
    b2d = b_o.reshape(1, H)

    def body(fa_ref, am_ref, w1_ref, w2_ref, b_ref, o_ref):
        acc = jnp.dot(fa_ref[...], w1_ref[...], preferred_element_type=jnp.float32)
        acc = acc + jnp.dot(am_ref[...], w2_ref[...], preferred_element_type=jnp.float32)
        o_ref[...] = jnp.maximum(acc + b_ref[...], 0.0)

    return pl.pallas_call(
        body,
        grid=(M // bm,),
        in_specs=[pl.BlockSpec((bm, ATOM_FDIM), lambda i: (i, 0)),
                  pl.BlockSpec((bm, H), lambda i: (i, 0)),
                  pl.BlockSpec((ATOM_FDIM, H), lambda i: (0, 0)),
                  pl.BlockSpec((H, H), lambda i: (0, 0)),
                  pl.BlockSpec((1, H), lambda i: (0, 0))],
        out_specs=pl.BlockSpec((bm, H), lambda i: (i, 0)),
        out_shape=jax.ShapeDtypeStruct((M, H), jnp.float32),
    )(fa, am, W1, W2, b2d)


def _pad_idx(ix):
    return jnp.pad(ix, ((0, A_PAD - N_ATOMS), (0, 0))).reshape(-1)


def kernel(f_atoms, f_bonds, a2b, b2a, b2revb, a2a,
           W_i_a, W_h_a_0, W_h_a_1, W_o_a, b_o_a,
           W_i_b, W_h_b_0, W_h_b_1, W_o_b, b_o_b):
    a2a_f = _pad_idx(a2a.astype(jnp.int32))
    a2b_f = _pad_idx(a2b.astype(jnp.int32))
    b2a32 = b2a.astype(jnp.int32)
    b2revb32 = b2revb.astype(jnp.int32)

    gsm_atom = _make_gather_summax(N_ATOMS)
    gsm_bond = _make_gather_summax(N_BONDS)
    dual = _make_dual_gather_sub(N_ATOMS)

    inp_a, msg = _mm_input(f_atoms, W_i_a, 400)
    for Wh in (W_h_a_0, W_h_a_1):
        am = gsm_atom(msg, a2a_f)[:N_ATOMS]
        msg = _iter_step(am, inp_a, Wh, 400)
    am = gsm_atom(msg, a2a_f)[:N_ATOMS]
    atom_h = _out_layer(f_atoms, am, W_o_a, b_o_a, 400)
    atom_vecs = atom_h.reshape(N_MOLS, MOL_SIZE, H)

    inp_b, msg = _mm_input(f_bonds, W_i_b, 2560)
    for Wh in (W_h_b_0, W_h_b_1):
        am = gsm_bond(msg, a2b_f)[:N_ATOMS]
        d = dual(am, msg, b2a32, b2revb32)
        msg = _iter_step(d, inp_b, Wh, 2560)
    am = gsm_bond(msg, a2b_f)[:N_ATOMS]
    bond_h = _out_layer(f_atoms, am, W_o_b, b_o_b, 400)
    bond_vecs = bond_h.reshape(N_MOLS, MOL_SIZE, H)

    mask = jnp.ones((N_MOLS, MOL_SIZE), dtype=jnp.float32)
    return (atom_vecs, mask, bond_vecs, mask)


# speedup vs baseline: 1.0050x; 1.0039x over previous
import functools

import jax
import jax.numpy as jnp
from jax import lax
from jax.experimental import pallas as pl
from jax.experimental.pallas import tpu as pltpu
from jax.experimental.pallas import tpu_sc as plsc

N_ATOMS = 10000
N_BONDS = 320000
MAX_NB = 32
ATOM_FDIM = 128
BOND_FDIM = 144
H = 128
N_MOLS = 100
MOL_SIZE = 100

NC = 2
NS = 16
NW = NC * NS
L = 16

A_PAD = 10240


def _worker_id():
    return lax.axis_index("s") * NC + lax.axis_index("c")


def _make_gather_summax(n_src):
    RW = A_PAD // NW
    n_chunks = RW
    NB = 5
    n_outer = n_chunks // NB
    nl = H // L
    mesh = plsc.VectorSubcoreMesh(core_axis_name="c", subcore_axis_name="s")

    @functools.partial(
        pl.kernel,
        name=f"gsm_{n_src}",
        out_type=jax.ShapeDtypeStruct((A_PAD, H), jnp.float32),
        mesh=mesh,
        scratch_types=(
            [pltpu.VMEM((RW * MAX_NB,), jnp.int32)]
            + [pltpu.VMEM((RW, H), jnp.float32)]
            + [pltpu.VMEM((MAX_NB, H), jnp.float32) for _ in range(NB)]
            + [pltpu.SemaphoreType.DMA for _ in range(NB)]
        ),
    )
    def k(table_hbm, idx_hbm, out_hbm, idx_v, out_all, *bufs):
        rows = bufs[:NB]
        gsem = bufs[NB:2 * NB]
        wid = _worker_id()
        base = wid * RW

        pltpu.sync_copy(idx_hbm.at[pl.ds(base * MAX_NB, RW * MAX_NB)], idx_v)

        def start_gather(c, b):
            pltpu.async_copy(
                table_hbm.at[idx_v.at[pl.ds(c * MAX_NB, MAX_NB)]],
                rows[b], gsem[b])

        for b in range(NB):
            start_gather(b, b)

        def outer(i, carry):
            c0 = i * NB
            for b in range(NB):
                c = c0 + b
                pltpu.make_async_copy(
                    table_hbm.at[pl.ds(0, MAX_NB)], rows[b], gsem[b]
                ).wait()

                v0 = [rows[b][0, pl.ds(l * L, L)] for l in range(nl)]
                s_acc = list(v0)
                m_acc = list(v0)
                for r in range(1, MAX_NB):
                    vn = [rows[b][r, pl.ds(l * L, L)] for l in range(nl)]
                    s_acc = [s_acc[l] + vn[l] for l in range(nl)]
                    m_acc = [jnp.maximum(m_acc[l], vn[l]) for l in range(nl)]
                for l in range(nl):
                    out_all[c, pl.ds(l * L, L)] = s_acc[l] * m_acc[l]

                @pl.when(i + 1 < n_outer)
                def _():
                    start_gather(c - c0 + (i + 1) * NB, b)
            return carry

        lax.fori_loop(0, n_outer, outer, 0)
        pltpu.sync_copy(out_all, out_hbm.at[pl.ds(base, RW)])

    return k


def _make_dual_gather_sub(n_atab):
    CB = 40
    RW = N_BONDS // NW
    n_chunks = RW // CB
    NB = 5
    n_outer = n_chunks // NB
    mesh = plsc.VectorSubcoreMesh(core_axis_name="c", subcore_axis_name="s")

    @functools.partial(
        pl.kernel,
        name="dual_gather_sub",
        out_type=jax.ShapeDtypeStruct((N_BONDS, H), jnp.float32),
        mesh=mesh,
        scratch_types=(
            [pltpu.VMEM((RW,), jnp.int32) for _ in range(2)]
            + [pltpu.VMEM((CB, H), jnp.float32) for _ in range(3 * NB)]
            + [pltpu.SemaphoreType.DMA for _ in range(3 * NB)]
        ),
    )
    def k(amsg_hbm, msg_hbm, b2a_hbm, b2revb_hbm, out_hbm, ia_v, ir_v, *bufs):
        ra = bufs[0:NB]
        rr = bufs[NB:2 * NB]
        dv = bufs[2 * NB:3 * NB]
        sa = bufs[3 * NB:4 * NB]
        sr = bufs[4 * NB:5 * NB]
        sw = bufs[5 * NB:6 * NB]
        wid = _worker_id()
        base = wid * RW

        pltpu.sync_copy(b2a_hbm.at[pl.ds(base, RW)], ia_v)
        pltpu.sync_copy(b2revb_hbm.at[pl.ds(base, RW)], ir_v)
        for b in range(NB):
            pltpu.async_copy(amsg_hbm.at[ia_v.at[pl.ds(b * CB, CB)]], ra[b], sa[b])
            pltpu.async_copy(msg_hbm.at[ir_v.at[pl.ds(b * CB, CB)]], rr[b], sr[b])

        def outer(i, carry):
            c0 = i * NB
            for b in range(NB):
                c = c0 + b
                pltpu.make_async_copy(
                    amsg_hbm.at[pl.ds(0, CB)], ra[b], sa[b]).wait()
                pltpu.make_async_copy(
                    msg_hbm.at[pl.ds(0, CB)], rr[b], sr[b]).wait()

                @pl.when(i > 0)
                def _():
                    pltpu.make_async_copy(
                        dv[b], out_hbm.at[pl.ds(0, CB)], sw[b]).wait()

                def per_row(j, c2):
                    for l in range(H // L):
                        off = l * L
                        dv[b][j, pl.ds(off, L)] = (
                            ra[b][j, pl.ds(off, L)] - rr[b][j, pl.ds(off, L)])
                    return c2

                lax.fori_loop(0, CB, per_row, 0)
                pltpu.async_copy(dv[b], out_hbm.at[pl.ds(base + c * CB, CB)],
                                 sw[b])

                @pl.when(i + 1 < n_outer)
                def _():
                    cn = c - c0 + (i + 1) * NB
                    pltpu.async_copy(amsg_hbm.at[ia_v.at[pl.ds(cn * CB, CB)]],
                                     ra[b], sa[b])
                    pltpu.async_copy(msg_hbm.at[ir_v.at[pl.ds(cn * CB, CB)]],
                                     rr[b], sr[b])
            return carry

        lax.fori_loop(0, n_outer, outer, 0)
        for b in range(NB):
            pltpu.make_async_copy(dv[b], out_hbm.at[pl.ds(0, CB)],
                                  sw[b]).wait()

    return k


def _mm_input(x, W, bm):
    M, K = x.shape

    def body(x_ref, w_ref, inp_ref, msg_ref):
        acc = jnp.dot(x_ref[...], w_ref[...], preferred_element_type=jnp.float32)
        inp_ref[...] = acc
        msg_ref[...] = jnp.maximum(acc, 0.0)

    return pl.pallas_call(
        body,
        grid=(M // bm,),
        in_specs=[pl.BlockSpec((bm, K), lambda i: (i, 0)),
                  pl.BlockSpec((K, H), lambda i: (0, 0))],
        out_specs=[pl.BlockSpec((bm, H), lambda i: (i, 0)),
                   pl.BlockSpec((bm, H), lambda i: (i, 0))],
        out_shape=[jax.ShapeDtypeStruct((M, H), jnp.float32)] * 2,
    )(x, W)


def _iter_step(d, inp, Wh, bm):
    M = d.shape[0]

    def body(d_ref, i_ref, w_ref, o_ref):
        acc = jnp.dot(d_ref[...], w_ref[...], preferred_element_type=jnp.float32)
        o_ref[...] = jnp.maximum(i_ref[...] + acc, 0.0)

    return pl.pallas_call(
        body,
        grid=(M // bm,),
        in_specs=[pl.BlockSpec((bm, H), lambda i: (i, 0)),
                  pl.BlockSpec((bm, H), lambda i: (i, 0)),
                  pl.BlockSpec((H, H), lambda i: (0, 0))],
        out_specs=pl.BlockSpec((bm, H), lambda i: (i, 0)),
        out_shape=jax.ShapeDtypeStruct((M, H), jnp.float32),
    )(d, inp, Wh)


def _out_layer(fa, am, W_o, b_o, bm):
    M = fa.shape[0]
    W1 = W_o[:ATOM_FDIM]
    W2 = W_o[ATOM_FDIM:]
    b2d = b_o.reshape(1, H)

    def body(fa_ref, am_ref, w1_ref, w2_ref, b_ref, o_ref):
        acc = jnp.dot(fa_ref[...], w1_ref[...], preferred_element_type=jnp.float32)
        acc = acc + jnp.dot(am_ref[...], w2_ref[...], preferred_element_type=jnp.float32)
        o_ref[...] = jnp.maximum(acc + b_ref[...], 0.0)

    return pl.pallas_call(
        body,
        grid=(M // bm,),
        in_specs=[pl.BlockSpec((bm, ATOM_FDIM), lambda i: (i, 0)),
                  pl.BlockSpec((bm, H), lambda i: (i, 0)),
                  pl.BlockSpec((ATOM_FDIM, H), lambda i: (0, 0)),
                  pl.BlockSpec((H, H), lambda i: (0, 0)),
                  pl.BlockSpec((1, H), lambda i: (0, 0))],
        out_specs=pl.BlockSpec((bm, H), lambda i: (i, 0)),
        out_shape=jax.ShapeDtypeStruct((M, H), jnp.float32),
    )(fa, am, W1, W2, b2d)


def _pad_idx(ix):
    return jnp.pad(ix, ((0, A_PAD - N_ATOMS), (0, 0))).reshape(-1)


def kernel(f_atoms, f_bonds, a2b, b2a, b2revb, a2a,
           W_i_a, W_h_a_0, W_h_a_1, W_o_a, b_o_a,
           W_i_b, W_h_b_0, W_h_b_1, W_o_b, b_o_b):
    a2a_f = _pad_idx(a2a.astype(jnp.int32))
    a2b_f = _pad_idx(a2b.astype(jnp.int32))
    b2a32 = b2a.astype(jnp.int32)
    b2revb32 = b2revb.astype(jnp.int32)

    gsm_atom = _make_gather_summax(N_ATOMS)
    gsm_bond = _make_gather_summax(N_BONDS)
    dual = _make_dual_gather_sub(N_ATOMS)

    inp_a, msg = _mm_input(f_atoms, W_i_a, 400)
    for Wh in (W_h_a_0, W_h_a_1):
        am = gsm_atom(msg, a2a_f)[:N_ATOMS]
        msg = _iter_step(am, inp_a, Wh, 400)
    am = gsm_atom(msg, a2a_f)[:N_ATOMS]
    atom_h = _out_layer(f_atoms, am, W_o_a, b_o_a, 400)
    atom_vecs = atom_h.reshape(N_MOLS, MOL_SIZE, H)

    inp_b, msg = _mm_input(f_bonds, W_i_b, 2560)
    for Wh in (W_h_b_0, W_h_b_1):
        am = gsm_bond(msg, a2b_f)[:N_ATOMS]
        d = dual(am, msg, b2a32, b2revb32)
        msg = _iter_step(d, inp_b, Wh, 2560)
    am = gsm_bond(msg, a2b_f)[:N_ATOMS]
    bond_h = _out_layer(f_atoms, am, W_o_b, b_o_b, 400)
    bond_vecs = bond_h.reshape(N_MOLS, MOL_SIZE, H)

    mask = jnp.ones((N_MOLS, MOL_SIZE), dtype=jnp.float32)
    return (atom_vecs, mask, bond_vecs, mask)
